# Initial kernel scaffold; baseline (speedup 1.0000x reference)
import jax
import jax.numpy as jnp
from jax.experimental import pallas as pl


def kernel(h, tour, msg_w1, msg_b1, msg_w2, msg_b2, upd_w1, upd_b1, upd_w2, upd_b2, ln_g, ln_b):
    raise NotImplementedError("write your pallas kernel here")


# speedup vs baseline: 5.2113x; 5.2113x over previous
import functools

import jax
import jax.numpy as jnp
from jax import lax
from jax.experimental import pallas as pl
from jax.experimental.pallas import tpu as pltpu
from jax.experimental.pallas import tpu_sc as plsc

_NC = 2
_NS = 16
_NW = _NC * _NS
_CHUNK = 128


def _make_sc_gather(n_out_rows, n_table_rows, d):
    n_chunks = n_out_rows // _CHUNK
    k_max = (n_chunks + _NW - 1) // _NW
    mesh = plsc.VectorSubcoreMesh(core_axis_name="c", subcore_axis_name="s")

    @functools.partial(
        pl.kernel,
        mesh=mesh,
        out_type=jax.ShapeDtypeStruct((n_out_rows, d), jnp.float32),
        scratch_types=[
            pltpu.VMEM((1, _CHUNK), jnp.int32),
            pltpu.VMEM((_CHUNK, d), jnp.float32),
            pltpu.SemaphoreType.DMA,
        ],
    )
    def gather(table_hbm, idx_hbm, out_hbm, idx_v, rows_v, sem):
        w = lax.axis_index("s") * _NC + lax.axis_index("c")

        def body(k, carry):
            c = w + k * _NW

            @pl.when(c < n_chunks)
            def _():
                pltpu.sync_copy(idx_hbm.at[pl.ds(c * _CHUNK, _CHUNK)], idx_v.at[0])
                pltpu.async_copy(table_hbm.at[idx_v.at[0]], rows_v, sem).wait()
                pltpu.sync_copy(rows_v, out_hbm.at[pl.ds(c * _CHUNK, _CHUNK)])

            return carry

        lax.fori_loop(0, k_max, body, 0)

    return gather


def _make_sc_scatter(n_rows, d):
    n_chunks = n_rows // _CHUNK
    k_max = (n_chunks + _NW - 1) // _NW
    mesh = plsc.VectorSubcoreMesh(core_axis_name="c", subcore_axis_name="s")

    @functools.partial(
        pl.kernel,
        mesh=mesh,
        out_type=jax.ShapeDtypeStruct((n_rows, d), jnp.float32),
        scratch_types=[
            pltpu.VMEM((1, _CHUNK), jnp.int32),
            pltpu.VMEM((_CHUNK, d), jnp.float32),
            pltpu.SemaphoreType.DMA,
        ],
    )
    def scatter(vals_hbm, idx_hbm, out_hbm, idx_v, rows_v, sem):
        w = lax.axis_index("s") * _NC + lax.axis_index("c")

        def body(k, carry):
            c = w + k * _NW

            @pl.when(c < n_chunks)
            def _():
                pltpu.sync_copy(idx_hbm.at[pl.ds(c * _CHUNK, _CHUNK)], idx_v.at[0])
                pltpu.sync_copy(vals_hbm.at[pl.ds(c * _CHUNK, _CHUNK)], rows_v)
                pltpu.async_copy(rows_v, out_hbm.at[idx_v.at[0]], sem).wait()

            return carry

        lax.fori_loop(0, k_max, body, 0)

    return scatter


def _make_tc_body(T, D):
    def body(xp_ref, w1a_ref, w1b_ref, w2_ref, b1_ref, b2_ref,
             u1a_ref, u1b_ref, u2_ref, ub1_ref, ub2_ref,
             g_ref, beta_ref, out_ref):
        t = pl.program_id(1)
        f32 = jnp.float32
        A = xp_ref[0, pl.ds(t * T, T + 2), :]
        f_h = jnp.dot(A, w1a_ref[...], preferred_element_type=f32)
        g_h = jnp.dot(A, w1b_ref[...], preferred_element_type=f32)
        xc = A[1:T + 1]
        f = f_h[1:T + 1]
        b1 = b1_ref[0]
        yp = f + g_h[0:T] + b1
        yn = f + g_h[2:T + 2] + b1
        s = yp * jax.nn.sigmoid(yp) + yn * jax.nn.sigmoid(yn)
        msg = jnp.dot(s, w2_ref[...], preferred_element_type=f32) + 2.0 * b2_ref[0]
        u = (jnp.dot(xc, u1a_ref[...], preferred_element_type=f32)
             + jnp.dot(msg, u1b_ref[...], preferred_element_type=f32)
             + ub1_ref[0])
        u = u * jax.nn.sigmoid(u)
        upd = jnp.dot(u, u2_ref[...], preferred_element_type=f32) + ub2_ref[0]
        r = xc + upd
        mu = jnp.mean(r, axis=-1, keepdims=True)
        var = jnp.mean((r - mu) ** 2, axis=-1, keepdims=True)
        out_ref[0] = (r - mu) * lax.rsqrt(var + 1e-5) * g_ref[0] + beta_ref[0]

    return body


def _tc_compute(xp, weights, B, N, D, T):
    NT = N // T
    P = xp.shape[1]
    (w1a, w1b, w2, b1, b2, u1a, u1b, u2, ub1, ub2, g, beta) = weights

    def wspec(shape):
        return pl.BlockSpec(shape, lambda b, t: (0,) * len(shape))

    return pl.pallas_call(
        _make_tc_body(T, D),
        grid=(B, NT),
        in_specs=[
            pl.BlockSpec((1, P, D), lambda b, t: (b, 0, 0)),
            wspec((D, D)), wspec((D, D)), wspec((D, D)),
            wspec((1, D)), wspec((1, D)),
            wspec((D, D)), wspec((D, D)), wspec((D, D)),
            wspec((1, D)), wspec((1, D)),
            wspec((1, D)), wspec((1, D)),
        ],
        out_specs=pl.BlockSpec((1, T, D), lambda b, t: (b, t, 0)),
        out_shape=jax.ShapeDtypeStruct((B, N, D), jnp.float32),
        compiler_params=pltpu.CompilerParams(
            dimension_semantics=("arbitrary", "arbitrary"),
        ),
    )(xp, w1a, w1b, w2, b1, b2, u1a, u1b, u2, ub1, ub2, g, beta)


def kernel(h, tour, msg_w1, msg_b1, msg_w2, msg_b2,
           upd_w1, upd_b1, upd_w2, upd_b2, ln_g, ln_b):
    B, N, D = h.shape
    T = 1000
    P = N + 16

    base = (jnp.arange(B, dtype=jnp.int32) * N)[:, None]
    ext = jnp.concatenate(
        [tour[:, -1:], tour, tour[:, :1],
         jnp.zeros((B, P - N - 2), jnp.int32)], axis=1) + base
    h_flat = h.reshape(B * N, D)

    xp_flat = _make_sc_gather(B * P, B * N, D)(h_flat, ext.reshape(-1))
    xp = xp_flat.reshape(B, P, D)

    weights = (
        msg_w1[:D], msg_w1[D:], msg_w2,
        msg_b1.reshape(1, D), msg_b2.reshape(1, D),
        upd_w1[:D], upd_w1[D:], upd_w2,
        upd_b1.reshape(1, D), upd_b2.reshape(1, D),
        ln_g.reshape(1, D), ln_b.reshape(1, D),
    )
    h_new_pos = _tc_compute(xp, weights, B, N, D, T)

    flat_tour = (tour + base).reshape(-1)
    h_new_flat = _make_sc_scatter(B * N, D)(h_new_pos.reshape(B * N, D), flat_tour)
    return h_new_flat.reshape(B, N, D)
